# baseline (speedup 1.0000x reference)
import functools

import jax
import jax.numpy as jnp
from jax import lax
from jax.experimental import pallas as pl
from jax.experimental.pallas import tpu as pltpu
from jax.experimental.pallas import tpu_sc as plsc

_N_EMB = 1024
_D = 256
_B = 32768
_BM = 8192
_NB = _B // _BM
_COMMITMENT_COST = 0.25

_NC = 2
_NS = 16
_NW = _NC * _NS
_B_PER_W = _B // _NW
_CH = 128
_NCH = _B_PER_W // _CH


def _tc_body(x_ref, e_ref, en_ref, idx_ref, acc_ref):
    step = pl.program_id(0)
    x = x_ref[...]
    e = e_ref[...]
    s2 = lax.dot_general(x, e + e, (((1,), (1,)), ((), ())),
                         preferred_element_type=jnp.float32,
                         precision=lax.Precision.DEFAULT)
    xsq = jnp.sum(x * x, axis=1, keepdims=True)
    t = en_ref[...] + xsq
    dist = t - s2
    m = jnp.min(dist, axis=1, keepdims=True)
    jf = lax.broadcasted_iota(jnp.int32, (_BM, _N_EMB), 1).astype(jnp.float32)
    idxf = jnp.min(jnp.where(dist == m, jf, jnp.float32(_N_EMB)), axis=1)
    idx_ref[0, 0, :] = idxf.astype(jnp.int32)

    npad = (xsq > 0.0).astype(jnp.float32)
    lsum = jnp.sum(npad * m) / jnp.float32(_D)
    nsum = jnp.sum(npad)

    @pl.when(step == 0)
    def _():
        acc_ref[...] = jnp.zeros_like(acc_ref)

    lane = lax.broadcasted_iota(jnp.int32, (1, 128), 1)
    acc_ref[...] += jnp.where(lane == 0, lsum, 0.0) + jnp.where(lane == 1, nsum, 0.0)


def _tc_encode(x, embedding, en):
    return pl.pallas_call(
        _tc_body,
        grid=(_NB,),
        in_specs=[
            pl.BlockSpec((_BM, _D), lambda i: (i, 0)),
            pl.BlockSpec((_N_EMB, _D), lambda i: (0, 0)),
            pl.BlockSpec((1, _N_EMB), lambda i: (0, 0)),
        ],
        out_specs=[
            pl.BlockSpec((1, 1, _BM), lambda i: (i, 0, 0)),
            pl.BlockSpec((1, 128), lambda i: (0, 0)),
        ],
        out_shape=[
            jax.ShapeDtypeStruct((_NB, 1, _BM), jnp.int32),
            jax.ShapeDtypeStruct((1, 128), jnp.float32),
        ],
    )(x, embedding, en)


def _sc_gather_hist(embedding, idx3d):
    mesh = plsc.VectorSubcoreMesh(core_axis_name="c", subcore_axis_name="s")

    @functools.partial(
        pl.kernel,
        out_type=[
            jax.ShapeDtypeStruct((_B, _D), jnp.float32),
            jax.ShapeDtypeStruct((_NC, _N_EMB), jnp.float32),
        ],
        mesh=mesh,
        scratch_types=[
            pltpu.VMEM((_NCH, _CH), jnp.int32),
            pltpu.VMEM((_CH, _D), jnp.float32),
            pltpu.VMEM((_CH, _D), jnp.float32),
            pltpu.VMEM((_CH, _D), jnp.float32),
            pltpu.VMEM((_CH,), jnp.float32),
            pltpu.VMEM((_N_EMB,), jnp.float32),
            pltpu.VMEM_SHARED((_N_EMB,), jnp.float32),
            pltpu.SemaphoreType.DMA,
            pltpu.SemaphoreType.DMA,
            pltpu.SemaphoreType.DMA,
            pltpu.SemaphoreType.DMA,
            pltpu.SemaphoreType.DMA,
            pltpu.SemaphoreType.DMA,
        ],
    )
    def k(table_hbm, idx_hbm, out_hbm, cnt_hbm,
          idx_v, r0, r1, r2, ones_v, zb_v, bins_sh, g0, g1, g2, w0, w1, w2):
        cid = lax.axis_index("c")
        sid = lax.axis_index("s")
        wid = sid * _NC + cid
        base = wid * _B_PER_W
        pltpu.sync_copy(idx_hbm.at[wid], idx_v)

        @pl.when(sid == 0)
        def _():
            for i in range(_N_EMB // 16):
                zb_v[pl.ds(i * 16, 16)] = jnp.zeros((16,), jnp.float32)
            pltpu.sync_copy(zb_v, bins_sh)
        for i in range(_CH // 16):
            ones_v[pl.ds(i * 16, 16)] = jnp.ones((16,), jnp.float32)
        plsc.subcore_barrier()

        bufs = (r0, r1, r2)
        gsems = (g0, g1, g2)
        wsems = (w0, w1, w2)
        pend_w = [None, None, None]
        pend_g = [None, None, None]

        def issue_gather(c):
            b = c % 3
            if pend_w[b] is not None:
                pend_w[b].wait()
                pend_w[b] = None
            pend_g[b] = pltpu.async_copy(
                table_hbm.at[idx_v.at[c]], bufs[b], gsems[b])

        issue_gather(0)
        issue_gather(1)

        for c in range(_NCH):
            pltpu.sync_copy(ones_v, bins_sh.at[idx_v.at[c]], add=True)

        for c in range(_NCH):
            b = c % 3
            pend_g[b].wait()
            if c + 2 < _NCH:
                issue_gather(c + 2)
            pend_w[b] = pltpu.async_copy(
                bufs[b], out_hbm.at[pl.ds(base + c * _CH, _CH)], wsems[b])
        for w in pend_w:
            if w is not None:
                w.wait()

        plsc.subcore_barrier()

        @pl.when(sid == 0)
        def _():
            pltpu.sync_copy(bins_sh, cnt_hbm.at[cid])

    return k(embedding, idx3d)


def _fin_body(acc_ref, cnt_ref, stats_ref):
    lane = lax.broadcasted_iota(jnp.int32, (1, 128), 1)
    acc = acc_ref[...]
    loss_sum = jnp.sum(jnp.where(lane == 0, acc, 0.0))
    np_sum = jnp.sum(jnp.where(lane == 1, acc, 0.0))
    loss = _COMMITMENT_COST * loss_sum / np_sum
    cnt = cnt_ref[0:1, :] + cnt_ref[1:2, :]
    p = cnt / jnp.float32(_B)
    plx = jnp.exp(-jnp.sum(p * jnp.log(p + 1e-10)))
    stats_ref[...] = jnp.where(lane == 0, loss, 0.0) + jnp.where(lane == 1, plx, 0.0)


def _tc_finalize(acc, cnt2):
    return pl.pallas_call(
        _fin_body,
        out_shape=jax.ShapeDtypeStruct((1, 128), jnp.float32),
    )(acc, cnt2)


def kernel(x, embedding):
    en = jnp.sum(embedding ** 2, axis=1)[None, :]
    idx3, acc = _tc_encode(x, embedding, en)
    indices = idx3.reshape(_B)
    idx3d = idx3.reshape(_NW, _NCH, _CH)
    quantized, cnt2 = _sc_gather_hist(embedding, idx3d)
    stats = _tc_finalize(acc, cnt2)
    loss = stats[0, 0]
    perplexity = stats[0, 1]
    return (quantized, loss, indices, perplexity)


# speedup vs baseline: 1.0719x; 1.0719x over previous
import functools

import jax
import jax.numpy as jnp
from jax import lax
from jax.experimental import pallas as pl
from jax.experimental.pallas import tpu as pltpu
from jax.experimental.pallas import tpu_sc as plsc

_N_EMB = 1024
_D = 256
_B = 32768
_BM = 8192
_NB = _B // _BM
_COMMITMENT_COST = 0.25

_NC = 2
_NS = 16
_NW = _NC * _NS
_B_PER_W = _B // _NW
_CH = 128
_NCH = _B_PER_W // _CH


def _tc_body(x_ref, e_ref, en_ref, idx_ref, acc_ref):
    step = pl.program_id(0)
    x = x_ref[...]
    e = e_ref[...]
    s2 = lax.dot_general(x, e + e, (((1,), (1,)), ((), ())),
                         preferred_element_type=jnp.float32,
                         precision=lax.Precision.DEFAULT)
    xsq = jnp.sum(x * x, axis=1, keepdims=True)
    t = en_ref[...] + xsq
    dist = t - s2
    m = jnp.min(dist, axis=1, keepdims=True)
    ji = lax.broadcasted_iota(jnp.int32, (_BM, _N_EMB), 1)
    idx = jnp.min(jnp.where(dist == m, ji, _N_EMB), axis=1)
    idx_ref[0, 0, :] = idx

    npad = (xsq > 0.0).astype(jnp.float32)
    lsum = jnp.sum(npad * m) / jnp.float32(_D)
    nsum = jnp.sum(npad)

    @pl.when(step == 0)
    def _():
        acc_ref[...] = jnp.zeros_like(acc_ref)

    lane = lax.broadcasted_iota(jnp.int32, (1, 128), 1)
    acc_ref[...] += jnp.where(lane == 0, lsum, 0.0) + jnp.where(lane == 1, nsum, 0.0)


def _tc_encode(x, embedding, en):
    return pl.pallas_call(
        _tc_body,
        grid=(_NB,),
        in_specs=[
            pl.BlockSpec((_BM, _D), lambda i: (i, 0)),
            pl.BlockSpec((_N_EMB, _D), lambda i: (0, 0)),
            pl.BlockSpec((1, _N_EMB), lambda i: (0, 0)),
        ],
        out_specs=[
            pl.BlockSpec((1, 1, _BM), lambda i: (i, 0, 0)),
            pl.BlockSpec((1, 128), lambda i: (0, 0)),
        ],
        out_shape=[
            jax.ShapeDtypeStruct((_NB, 1, _BM), jnp.int32),
            jax.ShapeDtypeStruct((1, 128), jnp.float32),
        ],
    )(x, embedding, en)


def _sc_gather_hist(embedding, idx3d):
    mesh = plsc.VectorSubcoreMesh(core_axis_name="c", subcore_axis_name="s")

    @functools.partial(
        pl.kernel,
        out_type=[
            jax.ShapeDtypeStruct((_B, _D), jnp.float32),
            jax.ShapeDtypeStruct((_NC, _N_EMB), jnp.float32),
        ],
        mesh=mesh,
        scratch_types=[
            pltpu.VMEM((_NCH, _CH), jnp.int32),
            pltpu.VMEM((_CH, _D), jnp.float32),
            pltpu.VMEM((_CH, _D), jnp.float32),
            pltpu.VMEM((_CH, _D), jnp.float32),
            pltpu.VMEM((_CH,), jnp.float32),
            pltpu.VMEM((_N_EMB,), jnp.float32),
            pltpu.VMEM_SHARED((_N_EMB,), jnp.float32),
            pltpu.SemaphoreType.DMA,
            pltpu.SemaphoreType.DMA,
            pltpu.SemaphoreType.DMA,
            pltpu.SemaphoreType.DMA,
            pltpu.SemaphoreType.DMA,
            pltpu.SemaphoreType.DMA,
        ],
    )
    def k(table_hbm, idx_hbm, out_hbm, cnt_hbm,
          idx_v, r0, r1, r2, ones_v, zb_v, bins_sh, g0, g1, g2, w0, w1, w2):
        cid = lax.axis_index("c")
        sid = lax.axis_index("s")
        wid = sid * _NC + cid
        base = wid * _B_PER_W
        pltpu.sync_copy(idx_hbm.at[wid], idx_v)

        @pl.when(sid == 0)
        def _():
            for i in range(_N_EMB // 16):
                zb_v[pl.ds(i * 16, 16)] = jnp.zeros((16,), jnp.float32)
            pltpu.sync_copy(zb_v, bins_sh)
        for i in range(_CH // 16):
            ones_v[pl.ds(i * 16, 16)] = jnp.ones((16,), jnp.float32)
        plsc.subcore_barrier()

        bufs = (r0, r1, r2)
        gsems = (g0, g1, g2)
        wsems = (w0, w1, w2)
        pend_w = [None, None, None]
        pend_g = [None, None, None]

        def issue_gather(c):
            b = c % 3
            if pend_w[b] is not None:
                pend_w[b].wait()
                pend_w[b] = None
            pend_g[b] = pltpu.async_copy(
                table_hbm.at[idx_v.at[c]], bufs[b], gsems[b])

        issue_gather(0)
        issue_gather(1)

        for c in range(_NCH):
            pltpu.sync_copy(ones_v, bins_sh.at[idx_v.at[c]], add=True)

        for c in range(_NCH):
            b = c % 3
            pend_g[b].wait()
            if c + 2 < _NCH:
                issue_gather(c + 2)
            pend_w[b] = pltpu.async_copy(
                bufs[b], out_hbm.at[pl.ds(base + c * _CH, _CH)], wsems[b])
        for w in pend_w:
            if w is not None:
                w.wait()

        plsc.subcore_barrier()

        @pl.when(sid == 0)
        def _():
            pltpu.sync_copy(bins_sh, cnt_hbm.at[cid])

    return k(embedding, idx3d)


def _fin_body(acc_ref, cnt_ref, stats_ref):
    lane = lax.broadcasted_iota(jnp.int32, (1, 128), 1)
    acc = acc_ref[...]
    loss_sum = jnp.sum(jnp.where(lane == 0, acc, 0.0))
    np_sum = jnp.sum(jnp.where(lane == 1, acc, 0.0))
    loss = _COMMITMENT_COST * loss_sum / np_sum
    cnt = cnt_ref[0:1, :] + cnt_ref[1:2, :]
    p = cnt / jnp.float32(_B)
    plx = jnp.exp(-jnp.sum(p * jnp.log(p + 1e-10)))
    stats_ref[...] = jnp.where(lane == 0, loss, 0.0) + jnp.where(lane == 1, plx, 0.0)


def _tc_finalize(acc, cnt2):
    return pl.pallas_call(
        _fin_body,
        out_shape=jax.ShapeDtypeStruct((1, 128), jnp.float32),
    )(acc, cnt2)


def kernel(x, embedding):
    en = jnp.sum(embedding ** 2, axis=1)[None, :]
    idx3, acc = _tc_encode(x, embedding, en)
    indices = idx3.reshape(_B)
    idx3d = idx3.reshape(_NW, _NCH, _CH)
    quantized, cnt2 = _sc_gather_hist(embedding, idx3d)
    stats = _tc_finalize(acc, cnt2)
    loss = stats[0, 0]
    perplexity = stats[0, 1]
    return (quantized, loss, indices, perplexity)
